# Initial kernel scaffold; baseline (speedup 1.0000x reference)
import jax
import jax.numpy as jnp
from jax.experimental import pallas as pl


def kernel(x, emb_table, fc_w, fc_b):
    raise NotImplementedError("write your pallas kernel here")


# speedup vs baseline: 15.8356x; 15.8356x over previous
import functools

import jax
import jax.numpy as jnp
from jax import lax
from jax.experimental import pallas as pl
from jax.experimental.pallas import tpu as pltpu
from jax.experimental.pallas import tpu_sc as plsc

VOCAB = 100000
D = 64
NLANE = 16
NVREG = D // NLANE
B = 16384
L = 200
NCLS = 5

NC, NS = 2, 16
NW = NC * NS
RPW = B // NW
CH = 16
NCHUNK = RPW // CH

_SPLIT0 = 128
_SPLIT1 = L - _SPLIT0


def _pool_body(x_hbm, table_hbm, pooled_hbm, idx_v, buf_v, pooled_v, sem0, sem1):
    cid = lax.axis_index("c")
    sid = lax.axis_index("s")
    wid = sid * NC + cid
    row0 = wid * RPW

    def chunk_body(ci, _):
        chunk_row = row0 + ci * CH
        pltpu.sync_copy(x_hbm.at[pl.ds(chunk_row * L, CH * L)], idx_v)

        def row_body(j, _):
            off = j * L
            cp0 = pltpu.make_async_copy(
                table_hbm.at[idx_v.at[pl.ds(off, _SPLIT0)]],
                buf_v.at[pl.ds(0, _SPLIT0)], sem0)
            cp1 = pltpu.make_async_copy(
                table_hbm.at[idx_v.at[pl.ds(off + _SPLIT0, _SPLIT1)]],
                buf_v.at[pl.ds(_SPLIT0, _SPLIT1)], sem1)
            cp0.start()
            cp1.start()
            cp0.wait()
            cp1.wait()

            init = tuple(buf_v[0, pl.ds(c * NLANE, NLANE)] for c in range(NVREG))

            def red_body(i, accs):
                return tuple(
                    jnp.maximum(a, buf_v[i, pl.ds(c * NLANE, NLANE)])
                    for c, a in enumerate(accs))

            accs = lax.fori_loop(1, L, red_body, init, unroll=8)
            for c in range(NVREG):
                pooled_v[j, pl.ds(c * NLANE, NLANE)] = accs[c]
            return 0

        lax.fori_loop(0, CH, row_body, 0)
        pltpu.sync_copy(pooled_v, pooled_hbm.at[pl.ds(chunk_row, CH)])
        return 0

    lax.fori_loop(0, NCHUNK, chunk_body, 0)


_pool = functools.partial(
    pl.kernel,
    out_type=jax.ShapeDtypeStruct((B, D), jnp.float32),
    mesh=plsc.VectorSubcoreMesh(core_axis_name="c", subcore_axis_name="s"),
    scratch_types=[
        pltpu.VMEM((CH * L,), jnp.int32),
        pltpu.VMEM((L, D), jnp.float32),
        pltpu.VMEM((CH, D), jnp.float32),
        pltpu.SemaphoreType.DMA,
        pltpu.SemaphoreType.DMA,
    ],
    compiler_params=pltpu.CompilerParams(use_tc_tiling_on_sc=False),
)(_pool_body)


def _mm_body(p_ref, w_ref, b_ref, o_ref):
    o_ref[...] = lax.dot_general(
        p_ref[...], w_ref[...], (((1,), (1,)), ((), ())),
        preferred_element_type=jnp.float32) + b_ref[...]


_MM_BLK = 2048


def _linear(pooled, fc_w, fc_b2d):
    return pl.pallas_call(
        _mm_body,
        grid=(B // _MM_BLK,),
        in_specs=[
            pl.BlockSpec((_MM_BLK, D), lambda i: (i, 0)),
            pl.BlockSpec((NCLS, D), lambda i: (0, 0)),
            pl.BlockSpec((1, NCLS), lambda i: (0, 0)),
        ],
        out_specs=pl.BlockSpec((_MM_BLK, NCLS), lambda i: (i, 0)),
        out_shape=jax.ShapeDtypeStruct((B, NCLS), jnp.float32),
    )(pooled, fc_w, fc_b2d)


def kernel(x, emb_table, fc_w, fc_b):
    x_flat = x.reshape(B * L).astype(jnp.int32)
    pooled = _pool(x_flat, emb_table)
    return _linear(pooled, fc_w, fc_b.reshape(1, NCLS))


# speedup vs baseline: 30.5738x; 1.9307x over previous
import functools

import jax
import jax.numpy as jnp
from jax import lax
from jax.experimental import pallas as pl
from jax.experimental.pallas import tpu as pltpu
from jax.experimental.pallas import tpu_sc as plsc

VOCAB = 100000
D = 64
NLANE = 16
NVREG = D // NLANE
B = 16384
L = 200
NCLS = 5

NC, NS = 2, 16
NW = NC * NS
RPW = B // NW
CH = 16
NCHUNK = RPW // CH
RPG = 4
GPC = CH // RPG
GL = RPG * L
CHL = CH * L


def _pool_body(x_hbm, table_hbm, pooled_hbm, ibuf, gbuf, pooled_v, semi, semg):
    cid = lax.axis_index("c")
    sid = lax.axis_index("s")
    wid = sid * NC + cid
    row0 = wid * RPW

    def idx_start(ci, n):
        start = (row0 + ci * CH) * L
        pltpu.make_async_copy(
            x_hbm.at[pl.ds(start, CHL)], ibuf.at[n], semi.at[n]).start()

    def idx_wait(n):
        pltpu.make_async_copy(
            x_hbm.at[pl.ds(0, CHL)], ibuf.at[n], semi.at[n]).wait()

    def gather_start(k, p, q):
        off = k * GL
        pos = 0
        while pos < GL:
            n = min(128, GL - pos)
            pltpu.make_async_copy(
                table_hbm.at[ibuf.at[p, pl.ds(off + pos, n)]],
                gbuf.at[q, pl.ds(pos, n)], semg.at[q]).start()
            pos += n

    def gather_wait(q):
        pltpu.make_async_copy(
            table_hbm.at[pl.ds(0, GL)], gbuf.at[q], semg.at[q]).wait()

    def reduce_group(q, prow0):
        for r in range(RPG):
            base = r * L

            def body(i, accs, _base=base, _q=q):
                return tuple(
                    jnp.maximum(a, gbuf[_q, _base + i, pl.ds(c * NLANE, NLANE)])
                    for c, a in enumerate(accs))

            init = tuple(
                jnp.full((NLANE,), -jnp.inf, jnp.float32) for _ in range(NVREG))
            accs = lax.fori_loop(0, L, body, init, unroll=8)
            for c in range(NVREG):
                pooled_v[prow0 + r, pl.ds(c * NLANE, NLANE)] = accs[c]

    pltpu.sync_copy(x_hbm.at[pl.ds(row0 * L, CHL)], ibuf.at[0])
    idx_start(1, 1)
    gather_start(0, 0, 0)

    def pair_body(cp, _):
        for off in (0, 1):
            ci = cp * 2 + off
            p = off
            for k in range(GPC):
                q = k % 2
                gather_wait(q)
                if k < GPC - 1:
                    gather_start(k + 1, p, 1 - q)
                else:
                    @pl.when(ci + 1 < NCHUNK)
                    def _():
                        idx_wait(1 - p)
                        gather_start(0, 1 - p, 1 - q)

                    @pl.when(ci + 2 < NCHUNK)
                    def _():
                        idx_start(ci + 2, p)
                reduce_group(q, k * RPG)
            pltpu.sync_copy(pooled_v, pooled_hbm.at[pl.ds(row0 + ci * CH, CH)])
        return 0

    lax.fori_loop(0, NCHUNK // 2, pair_body, 0)


_pool = functools.partial(
    pl.kernel,
    out_type=jax.ShapeDtypeStruct((B, D), jnp.float32),
    mesh=plsc.VectorSubcoreMesh(core_axis_name="c", subcore_axis_name="s"),
    scratch_types=[
        pltpu.VMEM((2, CHL), jnp.int32),
        pltpu.VMEM((2, GL, D), jnp.float32),
        pltpu.VMEM((CH, D), jnp.float32),
        pltpu.SemaphoreType.DMA((2,)),
        pltpu.SemaphoreType.DMA((2,)),
    ],
    compiler_params=pltpu.CompilerParams(use_tc_tiling_on_sc=False),
)(_pool_body)


def _mm_body(p_ref, w_ref, b_ref, o_ref):
    o_ref[...] = lax.dot_general(
        p_ref[...], w_ref[...], (((1,), (1,)), ((), ())),
        preferred_element_type=jnp.float32) + b_ref[...]


_MM_BLK = 2048


def _linear(pooled, fc_w, fc_b2d):
    return pl.pallas_call(
        _mm_body,
        grid=(B // _MM_BLK,),
        in_specs=[
            pl.BlockSpec((_MM_BLK, D), lambda i: (i, 0)),
            pl.BlockSpec((NCLS, D), lambda i: (0, 0)),
            pl.BlockSpec((1, NCLS), lambda i: (0, 0)),
        ],
        out_specs=pl.BlockSpec((_MM_BLK, NCLS), lambda i: (i, 0)),
        out_shape=jax.ShapeDtypeStruct((B, NCLS), jnp.float32),
    )(pooled, fc_w, fc_b2d)


def kernel(x, emb_table, fc_w, fc_b):
    x_flat = x.reshape(B * L).astype(jnp.int32)
    pooled = _pool(x_flat, emb_table)
    return _linear(pooled, fc_w, fc_b.reshape(1, NCLS))


# speedup vs baseline: 35.4642x; 1.1600x over previous
import functools

import jax
import jax.numpy as jnp
from jax import lax
from jax.experimental import pallas as pl
from jax.experimental.pallas import tpu as pltpu
from jax.experimental.pallas import tpu_sc as plsc

VOCAB = 100000
D = 64
NLANE = 16
NVREG = D // NLANE
B = 16384
L = 200
NCLS = 5

NC, NS = 2, 16
NW = NC * NS
RPW = B // NW
CH = 16
NCHUNK = RPW // CH
RPG = 2
GPC = CH // RPG
GL = RPG * L
CHL = CH * L
NBUF = 4
AHEAD = NBUF - 1


def _pool_body(x_hbm, table_hbm, pooled_hbm, ibuf, gbuf, pooled_v, semi, semg):
    cid = lax.axis_index("c")
    sid = lax.axis_index("s")
    wid = sid * NC + cid
    row0 = wid * RPW

    def idx_start(ci, n):
        start = (row0 + ci * CH) * L
        pltpu.make_async_copy(
            x_hbm.at[pl.ds(start, CHL)], ibuf.at[n], semi.at[n]).start()

    def idx_wait(n):
        pltpu.make_async_copy(
            x_hbm.at[pl.ds(0, CHL)], ibuf.at[n], semi.at[n]).wait()

    def gather_start(k, p, q):
        off = k * GL
        pos = 0
        while pos < GL:
            n = min(128, GL - pos)
            pltpu.make_async_copy(
                table_hbm.at[ibuf.at[p, pl.ds(off + pos, n)]],
                gbuf.at[q, pl.ds(pos, n)], semg.at[q]).start()
            pos += n

    def gather_wait(q):
        pltpu.make_async_copy(
            table_hbm.at[pl.ds(0, GL)], gbuf.at[q], semg.at[q]).wait()

    def reduce_group(q, prow0):
        for r in range(RPG):
            base = r * L

            def body(i, accs, _base=base, _q=q):
                return tuple(
                    jnp.maximum(a, gbuf[_q, _base + i, pl.ds(c * NLANE, NLANE)])
                    for c, a in enumerate(accs))

            init = tuple(
                jnp.full((NLANE,), -jnp.inf, jnp.float32) for _ in range(NVREG))
            accs = lax.fori_loop(0, L, body, init, unroll=8)
            for c in range(NVREG):
                pooled_v[prow0 + r, pl.ds(c * NLANE, NLANE)] = accs[c]

    pltpu.sync_copy(x_hbm.at[pl.ds(row0 * L, CHL)], ibuf.at[0])
    idx_start(1, 1)
    for k in range(AHEAD):
        gather_start(k, 0, k % NBUF)

    def pair_body(cp, _):
        for offc in (0, 1):
            ci = cp * 2 + offc
            p = offc
            for k in range(GPC):
                q = k % NBUF
                gather_wait(q)
                kt = k + AHEAD
                qt = kt % NBUF
                if kt < GPC:
                    gather_start(kt, p, qt)
                elif kt == GPC:
                    @pl.when(ci + 1 < NCHUNK)
                    def _():
                        idx_wait(1 - p)
                        gather_start(0, 1 - p, qt)
                else:
                    @pl.when(ci + 1 < NCHUNK)
                    def _():
                        gather_start(kt - GPC, 1 - p, qt)
                if k == GPC - 1:
                    @pl.when(ci + 2 < NCHUNK)
                    def _():
                        idx_start(ci + 2, p)
                reduce_group(q, k * RPG)
            pltpu.sync_copy(pooled_v, pooled_hbm.at[pl.ds(row0 + ci * CH, CH)])
        return 0

    lax.fori_loop(0, NCHUNK // 2, pair_body, 0)


_pool = functools.partial(
    pl.kernel,
    out_type=jax.ShapeDtypeStruct((B, D), jnp.float32),
    mesh=plsc.VectorSubcoreMesh(core_axis_name="c", subcore_axis_name="s"),
    scratch_types=[
        pltpu.VMEM((2, CHL), jnp.int32),
        pltpu.VMEM((NBUF, GL, D), jnp.float32),
        pltpu.VMEM((CH, D), jnp.float32),
        pltpu.SemaphoreType.DMA((2,)),
        pltpu.SemaphoreType.DMA((NBUF,)),
    ],
    compiler_params=pltpu.CompilerParams(use_tc_tiling_on_sc=False),
)(_pool_body)


def _mm_body(p_ref, w_ref, b_ref, o_ref):
    o_ref[...] = lax.dot_general(
        p_ref[...], w_ref[...], (((1,), (1,)), ((), ())),
        preferred_element_type=jnp.float32) + b_ref[...]


_MM_BLK = 2048


def _linear(pooled, fc_w, fc_b2d):
    return pl.pallas_call(
        _mm_body,
        grid=(B // _MM_BLK,),
        in_specs=[
            pl.BlockSpec((_MM_BLK, D), lambda i: (i, 0)),
            pl.BlockSpec((NCLS, D), lambda i: (0, 0)),
            pl.BlockSpec((1, NCLS), lambda i: (0, 0)),
        ],
        out_specs=pl.BlockSpec((_MM_BLK, NCLS), lambda i: (i, 0)),
        out_shape=jax.ShapeDtypeStruct((B, NCLS), jnp.float32),
    )(pooled, fc_w, fc_b2d)


def kernel(x, emb_table, fc_w, fc_b):
    x_flat = x.reshape(B * L).astype(jnp.int32)
    pooled = _pool(x_flat, emb_table)
    return _linear(pooled, fc_w, fc_b.reshape(1, NCLS))


# speedup vs baseline: 35.7536x; 1.0082x over previous
import functools

import jax
import jax.numpy as jnp
from jax import lax
from jax.experimental import pallas as pl
from jax.experimental.pallas import tpu as pltpu
from jax.experimental.pallas import tpu_sc as plsc

VOCAB = 100000
D = 64
NLANE = 16
NVREG = D // NLANE
B = 16384
L = 200
NCLS = 5

NC, NS = 2, 16
NW = NC * NS
RPW = B // NW
CH = 16
NCHUNK = RPW // CH
RPG = 2
GPC = CH // RPG
GL = RPG * L
CHL = CH * L
NBUF = 4
AHEAD = NBUF - 1


def _body(x_hbm, table_hbm, w_hbm, b_hbm, out_hbm,
          ibuf, gbuf, out_v, wb_v, semi, semg):
    cid = lax.axis_index("c")
    sid = lax.axis_index("s")
    wid = sid * NC + cid
    row0 = wid * RPW

    iota = lax.iota(jnp.int32, NLANE)

    def idx_start(ci, n):
        start = (row0 + ci * CH) * L
        pltpu.make_async_copy(
            x_hbm.at[pl.ds(start, CHL)], ibuf.at[n], semi.at[n]).start()

    def idx_wait(n):
        pltpu.make_async_copy(
            x_hbm.at[pl.ds(0, CHL)], ibuf.at[n], semi.at[n]).wait()

    def gather_start(k, p, q):
        off = k * GL
        pos = 0
        while pos < GL:
            n = min(128, GL - pos)
            pltpu.make_async_copy(
                table_hbm.at[ibuf.at[p, pl.ds(off + pos, n)]],
                gbuf.at[q, pl.ds(pos, n)], semg.at[q]).start()
            pos += n

    def gather_wait(q):
        pltpu.make_async_copy(
            table_hbm.at[pl.ds(0, GL)], gbuf.at[q], semg.at[q]).wait()

    pltpu.sync_copy(w_hbm, wb_v.at[pl.ds(0, NCLS)])
    pltpu.sync_copy(b_hbm, wb_v.at[NCLS])
    wvec = [[wb_v[c, pl.ds(v * NLANE, NLANE)] for v in range(NVREG)]
            for c in range(NCLS)]
    bvec = wb_v[NCLS, pl.ds(0, NLANE)]

    def reduce_group(q, ci, k):
        for r in range(RPG):
            base = r * L

            def body(i, accs, _base=base, _q=q):
                return tuple(
                    jnp.maximum(a, gbuf[_q, _base + i, pl.ds(c * NLANE, NLANE)])
                    for c, a in enumerate(accs))

            init = tuple(
                jnp.full((NLANE,), -jnp.inf, jnp.float32) for _ in range(NVREG))
            accs = lax.fori_loop(0, L, body, init, unroll=8)
            orow = bvec
            for c in range(NCLS):
                t = accs[0] * wvec[c][0]
                for v in range(1, NVREG):
                    t = t + accs[v] * wvec[c][v]
                e = jnp.sum(t)
                orow = orow + jnp.where(iota == c, e, 0.0)
            out_v[ci * CH + k * RPG + r] = orow

    pltpu.sync_copy(x_hbm.at[pl.ds(row0 * L, CHL)], ibuf.at[0])
    idx_start(1, 1)
    for k in range(AHEAD):
        gather_start(k, 0, k % NBUF)

    def pair_body(cp, _):
        for offc in (0, 1):
            ci = cp * 2 + offc
            p = offc
            for k in range(GPC):
                q = k % NBUF
                gather_wait(q)
                kt = k + AHEAD
                qt = kt % NBUF
                if kt < GPC:
                    gather_start(kt, p, qt)
                elif kt == GPC:
                    @pl.when(ci + 1 < NCHUNK)
                    def _():
                        idx_wait(1 - p)
                        gather_start(0, 1 - p, qt)
                else:
                    @pl.when(ci + 1 < NCHUNK)
                    def _():
                        gather_start(kt - GPC, 1 - p, qt)
                if k == GPC - 1:
                    @pl.when(ci + 2 < NCHUNK)
                    def _():
                        idx_start(ci + 2, p)
                reduce_group(q, ci, k)
        return 0

    lax.fori_loop(0, NCHUNK // 2, pair_body, 0)
    pltpu.sync_copy(out_v, out_hbm.at[pl.ds(row0, RPW)])


_run = functools.partial(
    pl.kernel,
    out_type=jax.ShapeDtypeStruct((B, NLANE), jnp.float32),
    mesh=plsc.VectorSubcoreMesh(core_axis_name="c", subcore_axis_name="s"),
    scratch_types=[
        pltpu.VMEM((2, CHL), jnp.int32),
        pltpu.VMEM((NBUF, GL, D), jnp.float32),
        pltpu.VMEM((RPW, NLANE), jnp.float32),
        pltpu.VMEM((NCLS + 1, D), jnp.float32),
        pltpu.SemaphoreType.DMA((2,)),
        pltpu.SemaphoreType.DMA((NBUF,)),
    ],
    compiler_params=pltpu.CompilerParams(use_tc_tiling_on_sc=False, needs_layout_passes=False),
)(_body)


def kernel(x, emb_table, fc_w, fc_b):
    x_flat = x.reshape(B * L).astype(jnp.int32)
    b_pad = jnp.zeros((D,), jnp.float32).at[:NCLS].set(fc_b)
    out = _run(x_flat, emb_table, fc_w, b_pad)
    return out[:, :NCLS]
